# Initial kernel scaffold; baseline (speedup 1.0000x reference)
import jax
import jax.numpy as jnp
from jax.experimental import pallas as pl


def kernel(target, pre):
    raise NotImplementedError("write your pallas kernel here")


# speedup vs baseline: 1.3671x; 1.3671x over previous
import jax
import jax.numpy as jnp
from jax.experimental import pallas as pl
from jax.experimental.pallas import tpu as pltpu

_EPS = 0.05
_ITERS = 5
_N = 1024


def _loss_body(p_ref, tT_ref, out_ref):
    inv_eps = 1.0 / _EPS
    c = None
    for k in range(3):
        pk = p_ref[0, :, k:k + 1]
        tk = tT_ref[0, k:k + 1, :]
        d = pk - tk
        c = d * d if c is None else c + d * d

    dist1 = jnp.min(c, axis=1, keepdims=True)
    dist2 = jnp.min(c, axis=0, keepdims=True)
    cd = jnp.sum(dist1) / _N + 0.5 * (jnp.sum(dist2) / _N)

    log_a = -jnp.log(jnp.float32(_N))
    log_b = log_a
    g = jnp.zeros((1, _N), dtype=jnp.float32)
    f = jnp.zeros((_N, 1), dtype=jnp.float32)
    for _ in range(_ITERS):
        a = (g - c) * inv_eps
        m1 = jnp.max(a, axis=1, keepdims=True)
        lse1 = m1 + jnp.log(jnp.sum(jnp.exp(a - m1), axis=1, keepdims=True))
        f = _EPS * (log_a - lse1)
        b = (f - c) * inv_eps
        m2 = jnp.max(b, axis=0, keepdims=True)
        lse2 = m2 + jnp.log(jnp.sum(jnp.exp(b - m2), axis=0, keepdims=True))
        g = _EPS * (log_b - lse2)

    emd = jnp.sum(jnp.exp((f + g - c) * inv_eps) * c) * _N
    out_ref[...] = jnp.full((1, 1, 128), cd + emd, dtype=jnp.float32)


def kernel(target, pre):
    bsz = pre.shape[0]
    t_t = jnp.swapaxes(target, 1, 2)
    per_batch = pl.pallas_call(
        _loss_body,
        grid=(bsz,),
        in_specs=[
            pl.BlockSpec((1, _N, 3), lambda b: (b, 0, 0)),
            pl.BlockSpec((1, 3, _N), lambda b: (b, 0, 0)),
        ],
        out_specs=pl.BlockSpec((1, 1, 128), lambda b: (b, 0, 0)),
        out_shape=jax.ShapeDtypeStruct((bsz, 1, 128), jnp.float32),
    )(pre, t_t)
    return jnp.sum(per_batch[:, 0, 0])


# speedup vs baseline: 2.8489x; 2.0839x over previous
import jax
import jax.numpy as jnp
from jax.experimental import pallas as pl
from jax.experimental.pallas import tpu as pltpu

_EPS = 0.05
_ITERS = 5
_N = 1024
_TINY = 1e-30


def _loss_body(p_ref, tT_ref, out_ref):
    inv_eps = 1.0 / _EPS
    p = p_ref[0]
    t_t = tT_ref[0]
    pn = jnp.sum(p * p, axis=1, keepdims=True)
    tn = jnp.sum(t_t * t_t, axis=0, keepdims=True)
    cross = jax.lax.dot_general(
        p, t_t, (((1,), (0,)), ((), ())),
        preferred_element_type=jnp.float32)
    c = jnp.maximum(pn + tn - 2.0 * cross, 0.0)

    dist1 = jnp.min(c, axis=1, keepdims=True)
    dist2 = jnp.min(c, axis=0, keepdims=True)
    cd = jnp.sum(dist1) / _N + 0.5 * (jnp.sum(dist2) / _N)

    inv_n = jnp.float32(1.0 / _N)
    s_col = jnp.min(c - dist1, axis=0, keepdims=True)
    e = jnp.exp((dist1 + s_col - c) * inv_eps)
    w = jnp.exp(-s_col * inv_eps)
    v = jnp.zeros((_N, 1), dtype=jnp.float32)
    for _ in range(_ITERS):
        s = jnp.maximum(jnp.sum(e * w, axis=1, keepdims=True), _TINY)
        v = inv_n / s
        t_sum = jnp.maximum(jnp.sum(e * v, axis=0, keepdims=True), _TINY)
        w = inv_n / t_sum

    emd = jnp.sum(v * jnp.sum((e * c) * w, axis=1, keepdims=True)) * _N
    out_ref[...] = jnp.full((1, 1, 128), cd + emd, dtype=jnp.float32)


def kernel(target, pre):
    bsz = pre.shape[0]
    t_t = jnp.swapaxes(target, 1, 2)
    per_batch = pl.pallas_call(
        _loss_body,
        grid=(bsz,),
        in_specs=[
            pl.BlockSpec((1, _N, 3), lambda b: (b, 0, 0)),
            pl.BlockSpec((1, 3, _N), lambda b: (b, 0, 0)),
        ],
        out_specs=pl.BlockSpec((1, 1, 128), lambda b: (b, 0, 0)),
        out_shape=jax.ShapeDtypeStruct((bsz, 1, 128), jnp.float32),
    )(pre, t_t)
    return jnp.sum(per_batch[:, 0, 0])


# speedup vs baseline: 3.1358x; 1.1007x over previous
import jax
import jax.numpy as jnp
from jax.experimental import pallas as pl
from jax.experimental.pallas import tpu as pltpu

_EPS = 0.05
_ITERS = 5
_N = 1024
_TINY = 1e-30


def _loss_body(p_ref, t_ref, out_ref):
    inv_eps = jnp.float32(1.0 / _EPS)
    p = p_ref[0]
    t = t_ref[0]
    ones = jnp.ones((_N, 1), dtype=jnp.float32)
    pn = jnp.sum(p * p, axis=1, keepdims=True) * inv_eps
    tn = jnp.sum(t * t, axis=1, keepdims=True) * inv_eps
    p5 = jnp.concatenate([p * (-2.0 * inv_eps), pn, ones], axis=1)
    t5 = jnp.concatenate([t, ones, tn], axis=1)
    c20 = jax.lax.dot_general(
        p5, t5, (((1,), (1,)), ((), ())),
        preferred_element_type=jnp.float32)

    d1 = jnp.min(c20, axis=1, keepdims=True)
    d2 = jnp.min(c20, axis=0, keepdims=True)
    m = c20 - d1
    s20 = jnp.min(m, axis=0, keepdims=True)
    cd = (jnp.sum(d1) + 0.5 * jnp.sum(d2)) * jnp.float32(_EPS / _N)

    inv_n = jnp.float32(1.0 / _N)
    e = jnp.exp(s20 - m)
    w = jnp.exp(-s20)
    v = jnp.zeros((_N, 1), dtype=jnp.float32)
    for _ in range(_ITERS):
        s = jnp.maximum(jnp.sum(e * w, axis=1, keepdims=True), _TINY)
        v = inv_n / s
        t_sum = jnp.maximum(jnp.sum(e * v, axis=0, keepdims=True), _TINY)
        w = inv_n / t_sum

    emd = jnp.sum(v * jnp.sum((e * c20) * w, axis=1, keepdims=True))
    emd = emd * jnp.float32(_EPS * _N)
    out_ref[...] = jnp.full((1, 1, 128), cd + emd, dtype=jnp.float32)


def kernel(target, pre):
    bsz = pre.shape[0]
    per_batch = pl.pallas_call(
        _loss_body,
        grid=(bsz,),
        in_specs=[
            pl.BlockSpec((1, _N, 3), lambda b: (b, 0, 0)),
            pl.BlockSpec((1, _N, 3), lambda b: (b, 0, 0)),
        ],
        out_specs=pl.BlockSpec((1, 1, 128), lambda b: (b, 0, 0)),
        out_shape=jax.ShapeDtypeStruct((bsz, 1, 128), jnp.float32),
    )(pre, target)
    return jnp.sum(per_batch[:, 0, 0])
